# Initial kernel scaffold; baseline (speedup 1.0000x reference)
import jax
import jax.numpy as jnp
from jax.experimental import pallas as pl


def kernel(x, edge_index, batch, W1, b1, W2, b2, W3, b3, lin_W, lin_b):
    raise NotImplementedError("write your pallas kernel here")


# speedup vs baseline: 21.7913x; 21.7913x over previous
import functools

import numpy as np
import jax
import jax.numpy as jnp
from jax import lax
from jax.experimental import pallas as pl
from jax.experimental.pallas import tpu as pltpu
from jax.experimental.pallas import tpu_sc as plsc

N = 10000
E = 320000
D = 128
G = 64
NC = 2
NS = 16
NW = NC * NS
CHUNK = 128
KPT = 80
E_PAD = NW * KPT * CHUNK
N_PAD = 10112
RPT = N_PAD // NS
BN = 1000

_PAD_SRC = (np.arange(E_PAD - E, dtype=np.int32) * 197) % N
_PAD_DST = (N + np.arange(E_PAD - E, dtype=np.int32) % (N_PAD - N)).astype(np.int32)


def _deg_body(dstr, ones, zvec, out, dstv, ones_v, stage_v, dacc):
    cid = lax.axis_index("c")
    sid = lax.axis_index("s")
    wid = cid * NS + sid
    pltpu.sync_copy(zvec, stage_v)
    pltpu.sync_copy(stage_v, dacc.at[pl.ds(sid * RPT, RPT)])
    pltpu.sync_copy(ones, ones_v)
    pltpu.sync_copy(dstr.at[pl.ds(wid * KPT, KPT)], dstv)
    plsc.subcore_barrier()

    def body(j, carry):
        pltpu.sync_copy(ones_v, dacc.at[dstv.at[j]], add=True)
        return carry

    lax.fori_loop(0, KPT, body, 0)
    plsc.subcore_barrier()
    pltpu.sync_copy(dacc.at[pl.ds(sid * RPT, RPT)], stage_v)
    pltpu.sync_copy(stage_v, out.at[pl.ds(cid * N_PAD + sid * RPT, RPT)])


@functools.cache
def _sc_calls():
    mesh = plsc.VectorSubcoreMesh(
        core_axis_name="c", subcore_axis_name="s", num_cores=NC, num_subcores=NS
    )
    deg_call = pl.kernel(
        _deg_body,
        out_type=jax.ShapeDtypeStruct((NC * N_PAD,), jnp.float32),
        mesh=mesh,
        scratch_types=[
            pltpu.VMEM((KPT, CHUNK), jnp.int32),
            pltpu.VMEM((CHUNK,), jnp.float32),
            pltpu.VMEM((RPT,), jnp.float32),
            pltpu.VMEM_SHARED((N_PAD,), jnp.float32),
        ],
    )
    agg_call = pl.kernel(
        _agg_body,
        out_type=jax.ShapeDtypeStruct((NC, N_PAD, D), jnp.float32),
        mesh=mesh,
        scratch_types=[
            pltpu.VMEM((KPT // 2, CHUNK), jnp.int32),
            pltpu.VMEM((KPT // 2, CHUNK), jnp.int32),
            pltpu.VMEM((CHUNK, D), jnp.float32),
            pltpu.VMEM((CHUNK, D), jnp.float32),
            pltpu.VMEM_SHARED((N_PAD, D), jnp.float32),
            pltpu.SemaphoreType.DMA,
            pltpu.SemaphoreType.DMA,
        ],
    )
    return deg_call, agg_call


def _agg_body(hs, srcr, dstr, zrow, out, srcv, dstv, buf_a, buf_b, acc, sem_a, sem_b):
    cid = lax.axis_index("c")
    sid = lax.axis_index("s")
    wid = cid * NS + sid
    pltpu.sync_copy(zrow, buf_a)
    for k in range(4):
        pltpu.sync_copy(buf_a, acc.at[pl.ds(sid * RPT + k * CHUNK, CHUNK)])
    pltpu.sync_copy(
        buf_a.at[pl.ds(0, RPT - 4 * CHUNK)],
        acc.at[pl.ds(sid * RPT + 4 * CHUNK, RPT - 4 * CHUNK)],
    )
    plsc.subcore_barrier()

    for half in range(2):
        base = wid * KPT + half * (KPT // 2)
        pltpu.sync_copy(srcr.at[pl.ds(base, KPT // 2)], srcv)
        pltpu.sync_copy(dstr.at[pl.ds(base, KPT // 2)], dstv)

        def body(jj, carry):
            j = jj * 2
            cp_a = pltpu.async_copy(hs.at[srcv.at[j]], buf_a, sem_a)
            cp_b = pltpu.async_copy(hs.at[srcv.at[j + 1]], buf_b, sem_b)
            cp_a.wait()
            pltpu.sync_copy(buf_a, acc.at[dstv.at[j]], add=True)
            cp_b.wait()
            pltpu.sync_copy(buf_b, acc.at[dstv.at[j + 1]], add=True)
            return carry

        lax.fori_loop(0, KPT // 4, body, 0)
    plsc.subcore_barrier()
    for k in range(4):
        pltpu.sync_copy(acc.at[pl.ds(sid * RPT + k * CHUNK, CHUNK)], buf_a)
        pltpu.sync_copy(buf_a, out.at[cid, pl.ds(sid * RPT + k * CHUNK, CHUNK)])
    tail = RPT - 4 * CHUNK
    pltpu.sync_copy(
        acc.at[pl.ds(sid * RPT + 4 * CHUNK, tail)], buf_a.at[pl.ds(0, tail)]
    )
    pltpu.sync_copy(
        buf_a.at[pl.ds(0, tail)], out.at[cid, pl.ds(sid * RPT + 4 * CHUNK, tail)]
    )


def _mm1_body(x_ref, w_ref, degt_ref, hs_ref, dinv_ref):
    deg = 1.0 + degt_ref[:, 0:1] + degt_ref[:, 1:2]
    dinv = lax.rsqrt(deg)
    mm = jnp.dot(x_ref[...], w_ref[...], preferred_element_type=jnp.float32)
    hs_ref[...] = mm * dinv
    dinv_ref[...] = dinv


_mm1 = pl.pallas_call(
    _mm1_body,
    grid=(N // BN,),
    in_specs=[
        pl.BlockSpec((BN, D), lambda i: (i, 0)),
        pl.BlockSpec((D, D), lambda i: (0, 0)),
        pl.BlockSpec((BN, 2), lambda i: (i, 0)),
    ],
    out_specs=[
        pl.BlockSpec((BN, D), lambda i: (i, 0)),
        pl.BlockSpec((BN, 1), lambda i: (i, 0)),
    ],
    out_shape=[
        jax.ShapeDtypeStruct((N, D), jnp.float32),
        jax.ShapeDtypeStruct((N, 1), jnp.float32),
    ],
)


def _combine_body(p_ref, hsp_ref, dinv_ref, b_ref, w_ref, out_ref):
    s = p_ref[0] + p_ref[1] + hsp_ref[...]
    h = s * dinv_ref[...] + b_ref[...]
    h = jnp.maximum(h, 0.0)
    out_ref[...] = (
        jnp.dot(h, w_ref[...], preferred_element_type=jnp.float32) * dinv_ref[...]
    )


_combine = pl.pallas_call(
    _combine_body,
    grid=(N // BN,),
    in_specs=[
        pl.BlockSpec((NC, BN, D), lambda i: (0, i, 0)),
        pl.BlockSpec((BN, D), lambda i: (i, 0)),
        pl.BlockSpec((BN, 1), lambda i: (i, 0)),
        pl.BlockSpec((1, D), lambda i: (0, 0)),
        pl.BlockSpec((D, D), lambda i: (0, 0)),
    ],
    out_specs=pl.BlockSpec((BN, D), lambda i: (i, 0)),
    out_shape=jax.ShapeDtypeStruct((N, D), jnp.float32),
)


def _final_body(p_ref, hs_ref, dinv_ref, b_ref, batch_ref, lw_ref, lb_ref,
                out_ref, sums_ref, cnts_ref):
    i = pl.program_id(0)
    h3 = (p_ref[0] + p_ref[1] + hs_ref[...]) * dinv_ref[...] + b_ref[...]
    iota = lax.broadcasted_iota(jnp.int32, (BN, G), 1)
    mask = (batch_ref[...] == iota).astype(jnp.float32)
    psum = lax.dot_general(
        mask, h3, (((0,), (0,)), ((), ())), preferred_element_type=jnp.float32
    )
    pcnt = lax.dot_general(
        mask, jnp.ones_like(h3), (((0,), (0,)), ((), ())),
        preferred_element_type=jnp.float32,
    )

    @pl.when(i == 0)
    def _():
        sums_ref[...] = psum
        cnts_ref[...] = pcnt

    @pl.when(i > 0)
    def _():
        sums_ref[...] += psum
        cnts_ref[...] += pcnt

    @pl.when(i == N // BN - 1)
    def _():
        g = sums_ref[...] / jnp.maximum(cnts_ref[...], 1.0)
        out_ref[...] = (
            jnp.dot(g, lw_ref[...], preferred_element_type=jnp.float32) + lb_ref[...]
        )


_final = pl.pallas_call(
    _final_body,
    grid=(N // BN,),
    in_specs=[
        pl.BlockSpec((NC, BN, D), lambda i: (0, i, 0)),
        pl.BlockSpec((BN, D), lambda i: (i, 0)),
        pl.BlockSpec((BN, 1), lambda i: (i, 0)),
        pl.BlockSpec((1, D), lambda i: (0, 0)),
        pl.BlockSpec((BN, 1), lambda i: (i, 0)),
        pl.BlockSpec((D, 1), lambda i: (0, 0)),
        pl.BlockSpec((1, 1), lambda i: (0, 0)),
    ],
    out_specs=pl.BlockSpec((G, 1), lambda i: (0, 0)),
    out_shape=jax.ShapeDtypeStruct((G, 1), jnp.float32),
    scratch_shapes=[
        pltpu.VMEM((G, D), jnp.float32),
        pltpu.VMEM((G, D), jnp.float32),
    ],
)


def kernel(x, edge_index, batch, W1, b1, W2, b2, W3, b3, lin_W, lin_b):
    src = jnp.concatenate([edge_index[0], jnp.asarray(_PAD_SRC)]).reshape(
        E_PAD // CHUNK, CHUNK
    )
    dst = jnp.concatenate([edge_index[1], jnp.asarray(_PAD_DST)]).reshape(
        E_PAD // CHUNK, CHUNK
    )
    zvec = jnp.zeros((RPT,), jnp.float32)
    zrow = jnp.zeros((CHUNK, D), jnp.float32)
    ones = jnp.ones((CHUNK,), jnp.float32)

    _deg_call, _agg_call = _sc_calls()
    degp = _deg_call(dst, ones, zvec).reshape(NC, N_PAD)
    degt = degp[:, :N].T

    hs1, dinvc = _mm1(x, W1, degt)
    p1 = _agg_call(hs1, src, dst, zrow)
    hs2 = _combine(p1, hs1, dinvc, b1.reshape(1, D), W2)
    p2 = _agg_call(hs2, src, dst, zrow)
    hs3 = _combine(p2, hs2, dinvc, b2.reshape(1, D), W3)
    p3 = _agg_call(hs3, src, dst, zrow)
    return _final(
        p3, hs3, dinvc, b3.reshape(1, D), batch.reshape(N, 1),
        lin_W, lin_b.reshape(1, 1),
    )
